# baseline (speedup 1.0000x reference)
import functools

import jax
import jax.numpy as jnp
from jax import lax
from jax.experimental import pallas as pl
from jax.experimental.pallas import tpu as pltpu
from jax.experimental.pallas import tpu_sc as plsc

N = 10000
E = 320000
D = 128

NC, NS = 2, 16
NW = NC * NS
K = 128
C = 80
EPT = C * K
E_PAD = NW * EPT
N_PAD = 10112
SLAB = N_PAD // NS
BLK = 1264
RING = 16
N_DEG = 10240
SLAB_DEG = N_DEG // NS
MROWS = (C + 1) * K

_mesh = plsc.VectorSubcoreMesh(core_axis_name="c", subcore_axis_name="s")


@functools.partial(
    pl.kernel,
    out_type=jax.ShapeDtypeStruct((NC, 2, N_DEG), jnp.float32),
    mesh=_mesh,
    scratch_types=[
        pltpu.VMEM((C, K), jnp.int32),
        pltpu.VMEM((K,), jnp.float32),
        pltpu.VMEM_SHARED((N_DEG,), jnp.float32),
        pltpu.VMEM_SHARED((N_DEG,), jnp.float32),
    ],
)
def _deg_kernel(src_hbm, dst_hbm, zeros1_hbm, out_hbm, idx_v, ones_v, do_sh, di_sh):
    cid = lax.axis_index("c")
    sid = lax.axis_index("s")
    w = cid * NS + sid
    sl = pl.ds(sid * SLAB_DEG, SLAB_DEG)
    pltpu.sync_copy(zeros1_hbm.at[pl.ds(0, SLAB_DEG)], do_sh.at[sl])
    pltpu.sync_copy(zeros1_hbm.at[pl.ds(0, SLAB_DEG)], di_sh.at[sl])
    for i in range(K // 16):
        ones_v[pl.ds(i * 16, 16)] = jnp.ones((16,), jnp.float32)
    plsc.subcore_barrier()

    pltpu.sync_copy(src_hbm.at[w], idx_v)

    @pl.loop(0, C)
    def _(c):
        pltpu.sync_copy(ones_v, do_sh.at[idx_v.at[c]], add=True)

    pltpu.sync_copy(dst_hbm.at[w], idx_v)

    @pl.loop(0, C)
    def _(c):
        pltpu.sync_copy(ones_v, di_sh.at[idx_v.at[c]], add=True)

    plsc.subcore_barrier()
    pltpu.sync_copy(do_sh.at[sl], out_hbm.at[cid, 0, sl])
    pltpu.sync_copy(di_sh.at[sl], out_hbm.at[cid, 1, sl])


@functools.partial(
    pl.kernel,
    out_type=[
        jax.ShapeDtypeStruct((NC, N_PAD, D), jnp.float32),
        jax.ShapeDtypeStruct((NC, NS, MROWS, D), jnp.float32),
    ],
    mesh=_mesh,
    scratch_types=[
        pltpu.VMEM((C + 8, K), jnp.int32),
        pltpu.VMEM((2 * RING, K), jnp.int32),
        pltpu.VMEM((2, K, D), jnp.float32),
        pltpu.VMEM_SHARED((N_PAD, D), jnp.float32),
        pltpu.SemaphoreType.DMA,
        pltpu.SemaphoreType.DMA,
        pltpu.SemaphoreType.DMA,
    ],
)
def _gs_kernel(h_hbm, src_hbm, dst_hbm, z2_hbm, out_hbm, msg_hbm,
               src_v, dst_rr, rows_v, tab_sh, sem0, sem1, sem2):
    cid = lax.axis_index("c")
    sid = lax.axis_index("s")
    sl = pl.ds(sid * SLAB, SLAB)
    pltpu.sync_copy(src_hbm.at[cid, sid], src_v)
    pltpu.sync_copy(h_hbm.at[sl], tab_sh.at[sl])
    plsc.subcore_barrier()

    pltpu.async_copy(tab_sh.at[src_v.at[0]], rows_v.at[0], sem0)

    @pl.loop(0, C, step=2)
    def _(c):
        for b in range(2):
            ch = c + b
            sem_cur, sem_nxt = (sem0, sem1) if b == 0 else (sem1, sem0)
            pltpu.async_copy(tab_sh.at[src_v.at[ch + 1]], rows_v.at[1 - b], sem_nxt)
            pltpu.make_async_copy(h_hbm.at[pl.ds(0, K)], rows_v.at[b], sem_cur).wait()
            pltpu.sync_copy(rows_v.at[b], msg_hbm.at[cid, sid, pl.ds(ch * K, K)])

    pltpu.make_async_copy(h_hbm.at[pl.ds(0, K)], rows_v.at[0], sem0).wait()

    plsc.subcore_barrier()
    pltpu.sync_copy(z2_hbm, tab_sh.at[sl])
    pltpu.sync_copy(dst_hbm.at[cid, sid, pl.ds(0, 2 * RING)], dst_rr)
    plsc.subcore_barrier()

    pltpu.async_copy(msg_hbm.at[cid, sid, pl.ds(0, K)], rows_v.at[0], sem0)

    @pl.loop(0, C, step=RING)
    def _(g0):
        @pl.when(g0 > 0)
        def _():
            half = pl.ds(lax.rem(g0, 2 * RING), RING)
            pltpu.make_async_copy(dst_hbm.at[cid, sid, half], dst_rr.at[half],
                                  sem2).wait()

        @pl.when(g0 < C - RING)
        def _():
            nxt = pl.ds(g0 + RING, RING)
            half = pl.ds(lax.rem(g0 + RING, 2 * RING), RING)
            pltpu.async_copy(dst_hbm.at[cid, sid, nxt], dst_rr.at[half], sem2)

        @pl.loop(0, RING, step=2)
        def _(c):
            for b in range(2):
                ch = g0 + c + b
                sem_cur, sem_nxt = (sem0, sem1) if b == 0 else (sem1, sem0)
                slot_c = lax.rem(ch, 2 * RING)
                pltpu.async_copy(msg_hbm.at[cid, sid, pl.ds((ch + 1) * K, K)],
                                 rows_v.at[1 - b], sem_nxt)
                pltpu.make_async_copy(h_hbm.at[pl.ds(0, K)], rows_v.at[b],
                                      sem_cur).wait()
                pltpu.sync_copy(rows_v.at[b], tab_sh.at[dst_rr.at[slot_c]], add=True)

    pltpu.make_async_copy(h_hbm.at[pl.ds(0, K)], rows_v.at[0], sem0).wait()

    plsc.subcore_barrier()
    pltpu.sync_copy(tab_sh.at[sl], out_hbm.at[cid, sl])


def _mm1_body(x_ref, w_ref, degp_ref, o_ref):
    dout = degp_ref[0, 0] + degp_ref[1, 0]
    nout = jnp.where(dout > 0, lax.rsqrt(dout), 0.0)
    h = jnp.dot(x_ref[...], w_ref[...], preferred_element_type=jnp.float32)
    o_ref[...] = h * nout


def _mid_body(aggp_ref, degp_ref, b1_ref, w2_ref, o_ref):
    agg = aggp_ref[0] + aggp_ref[1]
    din = degp_ref[0, 1] + degp_ref[1, 1]
    dout = degp_ref[0, 0] + degp_ref[1, 0]
    nin = jnp.where(din > 0, lax.rsqrt(din), 0.0)
    nout = jnp.where(dout > 0, lax.rsqrt(dout), 0.0)
    z = jnp.maximum(agg * nin + b1_ref[...], 0.0)
    o_ref[...] = jnp.dot(z, w2_ref[...], preferred_element_type=jnp.float32) * nout


def _final_body(aggp_ref, degp_ref, b2_ref, o_ref):
    agg = aggp_ref[0] + aggp_ref[1]
    din = degp_ref[0, 1] + degp_ref[1, 1]
    nin = jnp.where(din > 0, lax.rsqrt(din), 0.0)
    o_ref[...] = agg * nin + b2_ref[...]


def _mm1(x_pad, W1, degp_r):
    grid = (N_PAD // BLK,)
    return pl.pallas_call(
        _mm1_body,
        grid=grid,
        in_specs=[
            pl.BlockSpec((BLK, D), lambda i: (i, 0)),
            pl.BlockSpec((D, D), lambda i: (0, 0)),
            pl.BlockSpec((NC, 2, BLK, 1), lambda i: (0, 0, i, 0)),
        ],
        out_specs=pl.BlockSpec((BLK, D), lambda i: (i, 0)),
        out_shape=jax.ShapeDtypeStruct((N_PAD, D), jnp.float32),
    )(x_pad, W1, degp_r)


def _mid(aggp, degp_r, b1r, W2):
    grid = (N_PAD // BLK,)
    return pl.pallas_call(
        _mid_body,
        grid=grid,
        in_specs=[
            pl.BlockSpec((NC, BLK, D), lambda i: (0, i, 0)),
            pl.BlockSpec((NC, 2, BLK, 1), lambda i: (0, 0, i, 0)),
            pl.BlockSpec((1, D), lambda i: (0, 0)),
            pl.BlockSpec((D, D), lambda i: (0, 0)),
        ],
        out_specs=pl.BlockSpec((BLK, D), lambda i: (i, 0)),
        out_shape=jax.ShapeDtypeStruct((N_PAD, D), jnp.float32),
    )(aggp, degp_r, b1r, W2)


def _final(aggp, degp_r, b2r):
    B2 = 2000
    grid = (N // B2,)
    return pl.pallas_call(
        _final_body,
        grid=grid,
        in_specs=[
            pl.BlockSpec((NC, B2, D), lambda i: (0, i, 0)),
            pl.BlockSpec((NC, 2, B2, 1), lambda i: (0, 0, i, 0)),
            pl.BlockSpec((1, D), lambda i: (0, 0)),
        ],
        out_specs=pl.BlockSpec((B2, D), lambda i: (i, 0)),
        out_shape=jax.ShapeDtypeStruct((N, D), jnp.float32),
    )(aggp, degp_r, b2r)


def kernel(features, edge_index, W1, b1, W2, b2):
    src = edge_index[0].astype(jnp.int32)
    dst = edge_index[1].astype(jnp.int32)
    padw = EPT - E // NW
    junk = N + (jnp.arange(padw, dtype=jnp.int32) % (N_PAD - N))
    junk2 = jnp.tile(junk, (NW, 1))
    src3 = jnp.concatenate([src.reshape(NW, E // NW), junk2], axis=1).reshape(NW, C, K)
    dst3 = jnp.concatenate([dst.reshape(NW, E // NW), junk2], axis=1).reshape(NW, C, K)
    src4 = src3.reshape(NC, NS, C, K)
    dst4 = dst3.reshape(NC, NS, C, K)
    src4p = jnp.concatenate([src4, jnp.zeros((NC, NS, 8, K), jnp.int32)], axis=2)
    zeros1 = jnp.zeros((N_DEG,), jnp.float32)
    zeros2 = jnp.zeros((SLAB, D), jnp.float32)
    x_pad = jnp.pad(features, ((0, N_PAD - N), (0, 0)))

    degp = _deg_kernel(src3, dst3, zeros1)
    degp_r = degp.reshape(NC, 2, N_DEG, 1)

    h1 = _mm1(x_pad, W1, degp_r)
    agg1, _ = _gs_kernel(h1, src4p, dst4, zeros2)
    h2 = _mid(agg1, degp_r, b1.reshape(1, D), W2)
    agg2, _ = _gs_kernel(h2, src4p, dst4, zeros2)
    return _final(agg2, degp_r, b2.reshape(1, D))


# speedup vs baseline: 1.0186x; 1.0186x over previous
import functools

import jax
import jax.numpy as jnp
from jax import lax
from jax.experimental import pallas as pl
from jax.experimental.pallas import tpu as pltpu
from jax.experimental.pallas import tpu_sc as plsc

N = 10000
E = 320000
D = 128

NC, NS = 2, 16
NW = NC * NS
K = 128
C = 80
EPT = C * K
E_PAD = NW * EPT
N_PAD = 10112
SLAB = N_PAD // NS
BLK = 1264
RING = 16
N_DEG = 10240
SLAB_DEG = N_DEG // NS
MROWS = (C + 1) * K

_mesh = plsc.VectorSubcoreMesh(core_axis_name="c", subcore_axis_name="s")


@functools.partial(
    pl.kernel,
    out_type=jax.ShapeDtypeStruct((NC, 2, N_DEG), jnp.float32),
    mesh=_mesh,
    scratch_types=[
        pltpu.VMEM((C, K), jnp.int32),
        pltpu.VMEM((C, K), jnp.int32),
        pltpu.VMEM((K,), jnp.float32),
        pltpu.VMEM_SHARED((N_DEG,), jnp.float32),
        pltpu.VMEM_SHARED((N_DEG,), jnp.float32),
        pltpu.SemaphoreType.DMA,
        pltpu.SemaphoreType.DMA,
    ],
)
def _deg_kernel(src_hbm, dst_hbm, zeros1_hbm, out_hbm, src_v, dst_v, ones_v,
                do_sh, di_sh, sem_a, sem_b):
    cid = lax.axis_index("c")
    sid = lax.axis_index("s")
    w = cid * NS + sid
    sl = pl.ds(sid * SLAB_DEG, SLAB_DEG)
    pltpu.sync_copy(zeros1_hbm.at[pl.ds(0, SLAB_DEG)], do_sh.at[sl])
    pltpu.sync_copy(zeros1_hbm.at[pl.ds(0, SLAB_DEG)], di_sh.at[sl])
    for i in range(K // 16):
        ones_v[pl.ds(i * 16, 16)] = jnp.ones((16,), jnp.float32)
    pltpu.sync_copy(src_hbm.at[w], src_v)
    pltpu.sync_copy(dst_hbm.at[w], dst_v)
    plsc.subcore_barrier()

    pltpu.async_copy(ones_v, do_sh.at[src_v.at[0]], sem_a, add=True)
    pltpu.async_copy(ones_v, di_sh.at[dst_v.at[0]], sem_b, add=True)

    @pl.loop(1, C)
    def _(c):
        pltpu.async_copy(ones_v, do_sh.at[src_v.at[c]], sem_a, add=True)
        pltpu.async_copy(ones_v, di_sh.at[dst_v.at[c]], sem_b, add=True)
        pltpu.make_async_copy(zeros1_hbm.at[pl.ds(0, K)], ones_v, sem_a).wait()
        pltpu.make_async_copy(zeros1_hbm.at[pl.ds(0, K)], ones_v, sem_b).wait()

    pltpu.make_async_copy(zeros1_hbm.at[pl.ds(0, K)], ones_v, sem_a).wait()
    pltpu.make_async_copy(zeros1_hbm.at[pl.ds(0, K)], ones_v, sem_b).wait()

    plsc.subcore_barrier()
    pltpu.sync_copy(do_sh.at[sl], out_hbm.at[cid, 0, sl])
    pltpu.sync_copy(di_sh.at[sl], out_hbm.at[cid, 1, sl])


@functools.partial(
    pl.kernel,
    out_type=[
        jax.ShapeDtypeStruct((NC, N_PAD, D), jnp.float32),
        jax.ShapeDtypeStruct((NC, NS, MROWS, D), jnp.float32),
    ],
    mesh=_mesh,
    scratch_types=[
        pltpu.VMEM((C + 8, K), jnp.int32),
        pltpu.VMEM((2 * RING, K), jnp.int32),
        pltpu.VMEM((2, K, D), jnp.float32),
        pltpu.VMEM_SHARED((N_PAD, D), jnp.float32),
        pltpu.SemaphoreType.DMA,
        pltpu.SemaphoreType.DMA,
        pltpu.SemaphoreType.DMA,
    ],
)
def _gs_kernel(h_hbm, src_hbm, dst_hbm, z2_hbm, out_hbm, msg_hbm,
               src_v, dst_rr, rows_v, tab_sh, sem0, sem1, sem2):
    cid = lax.axis_index("c")
    sid = lax.axis_index("s")
    sl = pl.ds(sid * SLAB, SLAB)
    pltpu.sync_copy(src_hbm.at[cid, sid], src_v)
    pltpu.sync_copy(h_hbm.at[sl], tab_sh.at[sl])
    plsc.subcore_barrier()

    pltpu.async_copy(tab_sh.at[src_v.at[0]], rows_v.at[0], sem0)

    @pl.loop(0, C, step=2)
    def _(c):
        for b in range(2):
            ch = c + b
            sem_cur, sem_nxt = (sem0, sem1) if b == 0 else (sem1, sem0)
            pltpu.async_copy(tab_sh.at[src_v.at[ch + 1]], rows_v.at[1 - b], sem_nxt)
            pltpu.make_async_copy(h_hbm.at[pl.ds(0, K)], rows_v.at[b], sem_cur).wait()
            pltpu.sync_copy(rows_v.at[b], msg_hbm.at[cid, sid, pl.ds(ch * K, K)])

    pltpu.make_async_copy(h_hbm.at[pl.ds(0, K)], rows_v.at[0], sem0).wait()

    plsc.subcore_barrier()
    pltpu.sync_copy(z2_hbm, tab_sh.at[sl])
    pltpu.sync_copy(dst_hbm.at[cid, sid, pl.ds(0, 2 * RING)], dst_rr)
    plsc.subcore_barrier()

    pltpu.async_copy(msg_hbm.at[cid, sid, pl.ds(0, K)], rows_v.at[0], sem0)

    @pl.loop(0, C, step=RING)
    def _(g0):
        @pl.when(g0 > 0)
        def _():
            half = pl.ds(lax.rem(g0, 2 * RING), RING)
            pltpu.make_async_copy(dst_hbm.at[cid, sid, half], dst_rr.at[half],
                                  sem2).wait()

        @pl.when(g0 < C - RING)
        def _():
            nxt = pl.ds(g0 + RING, RING)
            half = pl.ds(lax.rem(g0 + RING, 2 * RING), RING)
            pltpu.async_copy(dst_hbm.at[cid, sid, nxt], dst_rr.at[half], sem2)

        @pl.loop(0, RING, step=2)
        def _(c):
            for b in range(2):
                ch = g0 + c + b
                sem_cur, sem_nxt = (sem0, sem1) if b == 0 else (sem1, sem0)
                slot_c = lax.rem(ch, 2 * RING)
                pltpu.async_copy(msg_hbm.at[cid, sid, pl.ds((ch + 1) * K, K)],
                                 rows_v.at[1 - b], sem_nxt)
                pltpu.make_async_copy(h_hbm.at[pl.ds(0, K)], rows_v.at[b],
                                      sem_cur).wait()
                pltpu.sync_copy(rows_v.at[b], tab_sh.at[dst_rr.at[slot_c]], add=True)

    pltpu.make_async_copy(h_hbm.at[pl.ds(0, K)], rows_v.at[0], sem0).wait()

    plsc.subcore_barrier()
    pltpu.sync_copy(tab_sh.at[sl], out_hbm.at[cid, sl])


def _mm1_body(x_ref, w_ref, degp_ref, o_ref):
    dout = degp_ref[0, 0] + degp_ref[1, 0]
    nout = jnp.where(dout > 0, lax.rsqrt(dout), 0.0)
    h = jnp.dot(x_ref[...], w_ref[...], preferred_element_type=jnp.float32)
    o_ref[...] = h * nout


def _mid_body(aggp_ref, degp_ref, b1_ref, w2_ref, o_ref):
    agg = aggp_ref[0] + aggp_ref[1]
    din = degp_ref[0, 1] + degp_ref[1, 1]
    dout = degp_ref[0, 0] + degp_ref[1, 0]
    nin = jnp.where(din > 0, lax.rsqrt(din), 0.0)
    nout = jnp.where(dout > 0, lax.rsqrt(dout), 0.0)
    z = jnp.maximum(agg * nin + b1_ref[...], 0.0)
    o_ref[...] = jnp.dot(z, w2_ref[...], preferred_element_type=jnp.float32) * nout


def _final_body(aggp_ref, degp_ref, b2_ref, o_ref):
    agg = aggp_ref[0] + aggp_ref[1]
    din = degp_ref[0, 1] + degp_ref[1, 1]
    nin = jnp.where(din > 0, lax.rsqrt(din), 0.0)
    o_ref[...] = agg * nin + b2_ref[...]


def _mm1(x_pad, W1, degp_r):
    grid = (N_PAD // BLK,)
    return pl.pallas_call(
        _mm1_body,
        grid=grid,
        in_specs=[
            pl.BlockSpec((BLK, D), lambda i: (i, 0)),
            pl.BlockSpec((D, D), lambda i: (0, 0)),
            pl.BlockSpec((NC, 2, BLK, 1), lambda i: (0, 0, i, 0)),
        ],
        out_specs=pl.BlockSpec((BLK, D), lambda i: (i, 0)),
        out_shape=jax.ShapeDtypeStruct((N_PAD, D), jnp.float32),
    )(x_pad, W1, degp_r)


def _mid(aggp, degp_r, b1r, W2):
    grid = (N_PAD // BLK,)
    return pl.pallas_call(
        _mid_body,
        grid=grid,
        in_specs=[
            pl.BlockSpec((NC, BLK, D), lambda i: (0, i, 0)),
            pl.BlockSpec((NC, 2, BLK, 1), lambda i: (0, 0, i, 0)),
            pl.BlockSpec((1, D), lambda i: (0, 0)),
            pl.BlockSpec((D, D), lambda i: (0, 0)),
        ],
        out_specs=pl.BlockSpec((BLK, D), lambda i: (i, 0)),
        out_shape=jax.ShapeDtypeStruct((N_PAD, D), jnp.float32),
    )(aggp, degp_r, b1r, W2)


def _final(aggp, degp_r, b2r):
    B2 = 2000
    grid = (N // B2,)
    return pl.pallas_call(
        _final_body,
        grid=grid,
        in_specs=[
            pl.BlockSpec((NC, B2, D), lambda i: (0, i, 0)),
            pl.BlockSpec((NC, 2, B2, 1), lambda i: (0, 0, i, 0)),
            pl.BlockSpec((1, D), lambda i: (0, 0)),
        ],
        out_specs=pl.BlockSpec((B2, D), lambda i: (i, 0)),
        out_shape=jax.ShapeDtypeStruct((N, D), jnp.float32),
    )(aggp, degp_r, b2r)


def kernel(features, edge_index, W1, b1, W2, b2):
    src = edge_index[0].astype(jnp.int32)
    dst = edge_index[1].astype(jnp.int32)
    padw = EPT - E // NW
    junk = N + (jnp.arange(padw, dtype=jnp.int32) % (N_PAD - N))
    junk2 = jnp.tile(junk, (NW, 1))
    src3 = jnp.concatenate([src.reshape(NW, E // NW), junk2], axis=1).reshape(NW, C, K)
    dst3 = jnp.concatenate([dst.reshape(NW, E // NW), junk2], axis=1).reshape(NW, C, K)
    src4 = src3.reshape(NC, NS, C, K)
    dst4 = dst3.reshape(NC, NS, C, K)
    src4p = jnp.concatenate([src4, jnp.zeros((NC, NS, 8, K), jnp.int32)], axis=2)
    zeros1 = jnp.zeros((N_DEG,), jnp.float32)
    zeros2 = jnp.zeros((SLAB, D), jnp.float32)
    x_pad = jnp.pad(features, ((0, N_PAD - N), (0, 0)))

    degp = _deg_kernel(src3, dst3, zeros1)
    degp_r = degp.reshape(NC, 2, N_DEG, 1)

    h1 = _mm1(x_pad, W1, degp_r)
    agg1, _ = _gs_kernel(h1, src4p, dst4, zeros2)
    h2 = _mid(agg1, degp_r, b1.reshape(1, D), W2)
    agg2, _ = _gs_kernel(h2, src4p, dst4, zeros2)
    return _final(agg2, degp_r, b2.reshape(1, D))
